# baseline (device time: 156636 ns/iter reference)
import jax
import jax.numpy as jnp
from jax import lax
from jax.experimental import pallas as pl
from jax.experimental.pallas import tpu as pltpu

N_DEV = 32


def kernel(A, B):
    m_per, k = A.shape
    _, n = B.shape
    A = A.astype(jnp.bfloat16)
    B = B.astype(jnp.bfloat16)

    def body(a_ref, b_ref, out_ref, comm_ref, send_sems, recv_sems):
        my = lax.axis_index("i")
        left = lax.rem(my - 1 + N_DEV, N_DEV)
        right = lax.rem(my + 1, N_DEV)

        barrier = pltpu.get_barrier_semaphore()
        for nbr in (left, right):
            pl.semaphore_signal(
                barrier, inc=1,
                device_id=(nbr,), device_id_type=pl.DeviceIdType.MESH,
            )
        pl.semaphore_wait(barrier, 2)

        comm_ref[0] = a_ref[...]

        def compute_slot(s):
            origin = lax.rem(my - s + N_DEV, N_DEV)
            acc = jnp.dot(
                comm_ref[s], b_ref[...], preferred_element_type=jnp.float32
            )
            out_ref[pl.ds(origin * m_per, m_per), :] = acc.astype(out_ref.dtype)

        for h in range(N_DEV - 1):
            rdma = pltpu.make_async_remote_copy(
                src_ref=comm_ref.at[h],
                dst_ref=comm_ref.at[h + 1],
                send_sem=send_sems.at[h],
                recv_sem=recv_sems.at[h],
                device_id=(right,),
                device_id_type=pl.DeviceIdType.MESH,
            )
            rdma.start()
            compute_slot(h)
            rdma.wait()

        compute_slot(N_DEV - 1)

    out_shape = jax.ShapeDtypeStruct((N_DEV * m_per, n), jnp.bfloat16)
    return pl.pallas_call(
        body,
        out_shape=out_shape,
        in_specs=[
            pl.BlockSpec(memory_space=pltpu.VMEM),
            pl.BlockSpec(memory_space=pltpu.VMEM),
        ],
        out_specs=pl.BlockSpec(memory_space=pltpu.VMEM),
        scratch_shapes=[
            pltpu.VMEM((N_DEV, m_per, k), jnp.bfloat16),
            pltpu.SemaphoreType.DMA((N_DEV - 1,)),
            pltpu.SemaphoreType.DMA((N_DEV - 1,)),
        ],
        compiler_params=pltpu.CompilerParams(collective_id=0),
    )(A, B)


# device time: 126194 ns/iter; 1.2412x vs baseline; 1.2412x over previous
import jax
import jax.numpy as jnp
from jax import lax
from jax.experimental import pallas as pl
from jax.experimental.pallas import tpu as pltpu

N_DEV = 32
R_STEPS = 16
L_STEPS = 15


def kernel(A, B):
    m_per, k = A.shape
    _, n = B.shape
    A = A.astype(jnp.bfloat16)
    B = B.astype(jnp.bfloat16)

    def body(a_ref, b_ref, out_ref,
             comm_ref, send_r, recv_r, send_l, recv_l):
        my = lax.axis_index("i")
        left = lax.rem(my - 1 + N_DEV, N_DEV)
        right = lax.rem(my + 1, N_DEV)

        barrier = pltpu.get_barrier_semaphore()
        for nbr in (left, right):
            pl.semaphore_signal(
                barrier, inc=1,
                device_id=(nbr,), device_id_type=pl.DeviceIdType.MESH,
            )
        pl.semaphore_wait(barrier, 2)

        comm_ref[0] = a_ref[...]

        def compute(slot, origin):
            acc = jnp.dot(
                comm_ref[slot], b_ref[...], preferred_element_type=jnp.float32
            )
            out_ref[pl.ds(origin * m_per, m_per), :] = acc.astype(out_ref.dtype)

        for s in range(R_STEPS):
            r = pltpu.make_async_remote_copy(
                src_ref=comm_ref.at[s],
                dst_ref=comm_ref.at[s + 1],
                send_sem=send_r.at[s],
                recv_sem=recv_r.at[s],
                device_id=(right,),
                device_id_type=pl.DeviceIdType.MESH,
            )
            r.start()
            l = None
            if s < L_STEPS:
                lsrc = 0 if s == 0 else 16 + s
                l = pltpu.make_async_remote_copy(
                    src_ref=comm_ref.at[lsrc],
                    dst_ref=comm_ref.at[17 + s],
                    send_sem=send_l.at[s],
                    recv_sem=recv_l.at[s],
                    device_id=(left,),
                    device_id_type=pl.DeviceIdType.MESH,
                )
                l.start()
            compute(s, lax.rem(my - s + N_DEV, N_DEV))
            if s >= 1:
                compute(16 + s, lax.rem(my + s, N_DEV))
            r.wait()
            if l is not None:
                l.wait()

        compute(16, lax.rem(my - 16 + N_DEV, N_DEV))

    out_shape = jax.ShapeDtypeStruct((N_DEV * m_per, n), jnp.bfloat16)
    return pl.pallas_call(
        body,
        out_shape=out_shape,
        in_specs=[
            pl.BlockSpec(memory_space=pltpu.VMEM),
            pl.BlockSpec(memory_space=pltpu.VMEM),
        ],
        out_specs=pl.BlockSpec(memory_space=pltpu.VMEM),
        scratch_shapes=[
            pltpu.VMEM((N_DEV, m_per, k), jnp.bfloat16),
            pltpu.SemaphoreType.DMA((R_STEPS,)),
            pltpu.SemaphoreType.DMA((R_STEPS,)),
            pltpu.SemaphoreType.DMA((L_STEPS,)),
            pltpu.SemaphoreType.DMA((L_STEPS,)),
        ],
        compiler_params=pltpu.CompilerParams(collective_id=0),
    )(A, B)


# device time: 91873 ns/iter; 1.7049x vs baseline; 1.3736x over previous
import jax
import jax.numpy as jnp
import numpy as np
from jax import lax
from jax.experimental import pallas as pl
from jax.experimental.pallas import tpu as pltpu

N_DEV = 32
R_STEPS = 16
L_STEPS = 15

_P16 = [
    (0, 0), (1, 0), (2, 0), (3, 0),
    (3, 1), (2, 1), (1, 1), (1, 2),
    (2, 2), (3, 2), (3, 3), (2, 3),
    (1, 3), (0, 3), (0, 2), (0, 1),
]
_RING_COORDS = [(0, y, z) for (y, z) in _P16] + [
    (1, y, z) for (y, z) in reversed(_P16)
]

_PLANE_ORDER = [(0, 0), (1, 0), (1, 1), (0, 1), (0, 2), (1, 2), (1, 3), (0, 3)]


def _logical_of(x, y, z):
    return z * 8 + _PLANE_ORDER.index((x, y))


_RING_ORDER = np.array(
    [_logical_of(*c) for c in _RING_COORDS], dtype=np.int32
)
_RING_POS = np.argsort(_RING_ORDER).astype(np.int32)

_SLOT_OFF = np.array(
    list(range(R_STEPS + 1)) + [-j for j in range(1, L_STEPS + 1)],
    dtype=np.int32,
)


def kernel(A, B):
    m_per, k = A.shape
    _, n = B.shape
    A = A.astype(jnp.bfloat16)
    B = B.astype(jnp.bfloat16)

    my = lax.axis_index("i")
    ring_order = jnp.asarray(_RING_ORDER)
    pos = jnp.asarray(_RING_POS)[my]
    nbrs = jnp.stack([
        ring_order[jnp.mod(pos - 1, N_DEV)],
        ring_order[jnp.mod(pos + 1, N_DEV)],
    ]).astype(jnp.int32)
    origins = ring_order[jnp.mod(pos - jnp.asarray(_SLOT_OFF), N_DEV)].astype(
        jnp.int32
    )

    def body(nbrs_ref, origins_ref, a_ref, b_ref, out_ref,
             comm_ref, send_r, recv_r, send_l, recv_l):
        left = nbrs_ref[0]
        right = nbrs_ref[1]

        barrier = pltpu.get_barrier_semaphore()
        for nbr in (left, right):
            pl.semaphore_signal(
                barrier, inc=1,
                device_id=(nbr,), device_id_type=pl.DeviceIdType.MESH,
            )
        pl.semaphore_wait(barrier, 2)

        comm_ref[0] = a_ref[...]

        def compute(slot):
            origin = origins_ref[slot]
            acc = jnp.dot(
                comm_ref[slot], b_ref[...], preferred_element_type=jnp.float32
            )
            out_ref[pl.ds(origin * m_per, m_per), :] = acc.astype(out_ref.dtype)

        for s in range(R_STEPS):
            r = pltpu.make_async_remote_copy(
                src_ref=comm_ref.at[s],
                dst_ref=comm_ref.at[s + 1],
                send_sem=send_r.at[s],
                recv_sem=recv_r.at[s],
                device_id=(right,),
                device_id_type=pl.DeviceIdType.MESH,
            )
            r.start()
            l = None
            if s < L_STEPS:
                lsrc = 0 if s == 0 else 16 + s
                l = pltpu.make_async_remote_copy(
                    src_ref=comm_ref.at[lsrc],
                    dst_ref=comm_ref.at[17 + s],
                    send_sem=send_l.at[s],
                    recv_sem=recv_l.at[s],
                    device_id=(left,),
                    device_id_type=pl.DeviceIdType.MESH,
                )
                l.start()
            compute(s)
            if s >= 1:
                compute(16 + s)
            r.wait()
            if l is not None:
                l.wait()

        compute(16)

    out_shape = jax.ShapeDtypeStruct((N_DEV * m_per, n), jnp.bfloat16)
    return pl.pallas_call(
        body,
        out_shape=out_shape,
        in_specs=[
            pl.BlockSpec(memory_space=pltpu.SMEM),
            pl.BlockSpec(memory_space=pltpu.SMEM),
            pl.BlockSpec(memory_space=pltpu.VMEM),
            pl.BlockSpec(memory_space=pltpu.VMEM),
        ],
        out_specs=pl.BlockSpec(memory_space=pltpu.VMEM),
        scratch_shapes=[
            pltpu.VMEM((N_DEV, m_per, k), jnp.bfloat16),
            pltpu.SemaphoreType.DMA((R_STEPS,)),
            pltpu.SemaphoreType.DMA((R_STEPS,)),
            pltpu.SemaphoreType.DMA((L_STEPS,)),
            pltpu.SemaphoreType.DMA((L_STEPS,)),
        ],
        compiler_params=pltpu.CompilerParams(collective_id=0),
    )(nbrs, origins, A, B)
